# baseline (speedup 1.0000x reference)
import jax
import jax.numpy as jnp
from jax.experimental import pallas as pl

IN_DIM = 128
OUT_DIM = 128
NUM_HEADS = 4


def _leaky_relu(v):
    return jnp.where(v > 0, v, 0.2 * v)


def _elu_kernel(agg_ref, out_ref):
    v = agg_ref[...]
    out_ref[...] = jnp.where(v > 0, v, jnp.exp(v) - 1.0)


def kernel(x, edge_index, attn_w, attn_b, fc_w, fc_b):
    src = edge_index[0]
    dst = edge_index[1]
    n = x.shape[0]
    fc_h = x @ fc_w.T + fc_b
    aggs = []
    for hidx in range(NUM_HEADS):
        w = attn_w[hidx]
        b = attn_b[hidx]
        a_left = x @ w[:IN_DIM]
        a_right = x @ w[IN_DIM:]
        score_fwd = _leaky_relu(a_left[src] + a_right[dst] + b)
        score_rev = _leaky_relu(a_left[dst] + a_right[src] + b)
        denom = jax.ops.segment_sum(score_fwd, src, num_segments=n) + \
                jax.ops.segment_sum(score_rev, dst, num_segments=n)
        alpha = score_fwd / denom[src]
        agg = jax.ops.segment_sum(alpha[:, None] * fc_h[dst], src, num_segments=n)
        aggs.append(agg)
    agg_all = jnp.concatenate(aggs, axis=1)
    return pl.pallas_call(
        _elu_kernel,
        out_shape=jax.ShapeDtypeStruct(agg_all.shape, agg_all.dtype),
    )(agg_all)


# speedup vs baseline: 13.5648x; 13.5648x over previous
import functools

import jax
import jax.numpy as jnp
from jax import lax
from jax.experimental import pallas as pl
from jax.experimental.pallas import tpu as pltpu
from jax.experimental.pallas import tpu_sc as plsc

N = 10000
E = 320000
F = 128
H = 4
FC = 32
NC = 2
NS = 16
NW = NC * NS
EB = E // NW
CH = 80
NCHUNK = EB // CH
G = CH // 16


def _iota16():
    return lax.iota(jnp.int32, 16)


def _full16(v):
    return jnp.full((16,), v, dtype=jnp.int32)


def _lrelu(v):
    return jnp.where(v > 0, v, 0.2 * v)


def _tc1_body(x_ref, fcw_ref, fcb_ref, w8_ref, b8_ref, fch_ref, snode_ref):
    xb = x_ref[...]
    fch_ref[...] = (
        jnp.dot(xb, fcw_ref[...].T, preferred_element_type=jnp.float32)
        + fcb_ref[...][None, :]
    )
    snode_ref[...] = (
        jnp.dot(xb, w8_ref[...], preferred_element_type=jnp.float32)
        + b8_ref[...][None, :]
    )


def _tc1(x, fc_w, fc_b, w8, b8):
    bn = 1000
    grid = (N // bn,)
    return pl.pallas_call(
        _tc1_body,
        grid=grid,
        in_specs=[
            pl.BlockSpec((bn, F), lambda i: (i, 0)),
            pl.BlockSpec((F, F), lambda i: (0, 0)),
            pl.BlockSpec((F,), lambda i: (0,)),
            pl.BlockSpec((F, 2 * H), lambda i: (0, 0)),
            pl.BlockSpec((2 * H,), lambda i: (0,)),
        ],
        out_specs=[
            pl.BlockSpec((bn, F), lambda i: (i, 0)),
            pl.BlockSpec((bn, 2 * H), lambda i: (i, 0)),
        ],
        out_shape=[
            jax.ShapeDtypeStruct((N, F), jnp.float32),
            jax.ShapeDtypeStruct((N, 2 * H), jnp.float32),
        ],
    )(x, fc_w, fc_b, w8, b8)


def _sc1_body(src_hbm, dst_hbm, snode_hbm, zerosn_hbm,
              sf_hbm, dpart_hbm,
              snode_v, src_v, dst_v, sfb, srb, d_sp):
    c = lax.axis_index("c")
    s = lax.axis_index("s")
    cwid = c * NS + s
    base_e = cwid * EB

    pltpu.sync_copy(snode_hbm, snode_v)

    @pl.when(s == 0)
    def _():
        for h in range(H):
            pltpu.sync_copy(zerosn_hbm, d_sp[h])

    plsc.subcore_barrier()

    def chunk_body(k, carry):
        e0 = base_e + k * CH
        pltpu.sync_copy(src_hbm.at[pl.ds(e0, CH)], src_v)
        pltpu.sync_copy(dst_hbm.at[pl.ds(e0, CH)], dst_v)
        for g in range(G):
            sb = src_v[pl.ds(g * 16, 16)] * 8
            db = dst_v[pl.ds(g * 16, 16)] * 8
            for h in range(H):
                al_s = plsc.load_gather(snode_v, [sb + h])
                ar_s = plsc.load_gather(snode_v, [sb + (H + h)])
                al_d = plsc.load_gather(snode_v, [db + h])
                ar_d = plsc.load_gather(snode_v, [db + (H + h)])
                sfb[h][pl.ds(g * 16, 16)] = _lrelu(al_s + ar_d)
                srb[h][pl.ds(g * 16, 16)] = _lrelu(al_d + ar_s)
        for h in range(H):
            pltpu.sync_copy(sfb[h], sf_hbm.at[pl.ds(h * E + e0, CH)])
            pltpu.sync_copy(sfb[h], d_sp[h].at[src_v], add=True)
            pltpu.sync_copy(srb[h], d_sp[h].at[dst_v], add=True)
        return carry

    lax.fori_loop(0, NCHUNK, chunk_body, 0)

    plsc.subcore_barrier()

    @pl.when(s == 0)
    def _():
        for h in range(H):
            pltpu.sync_copy(d_sp[h], dpart_hbm.at[c, h])


def _sc1(src, dst, snode_flat, zerosn):
    mesh = plsc.VectorSubcoreMesh(core_axis_name="c", subcore_axis_name="s")
    return pl.kernel(
        _sc1_body,
        out_type=[
            jax.ShapeDtypeStruct((H * E,), jnp.float32),
            jax.ShapeDtypeStruct((NC, H, N), jnp.float32),
        ],
        mesh=mesh,
        compiler_params=pltpu.CompilerParams(needs_layout_passes=False),
        scratch_types=[
            pltpu.VMEM((N * 2 * H,), jnp.float32),
            pltpu.VMEM((CH,), jnp.int32),
            pltpu.VMEM((CH,), jnp.int32),
            [pltpu.VMEM((CH,), jnp.float32) for _ in range(H)],
            [pltpu.VMEM((CH,), jnp.float32) for _ in range(H)],
            [pltpu.VMEM_SHARED((N,), jnp.float32) for _ in range(H)],
        ],
    )(src, dst, snode_flat, zerosn)


def _sc2_body(src_hbm, sf_hbm, dpart_hbm,
              alpha_hbm,
              dpart_v, src_v, sfc, alc):
    c = lax.axis_index("c")
    s = lax.axis_index("s")
    cwid = c * NS + s
    base_e = cwid * EB

    pltpu.sync_copy(dpart_hbm, dpart_v)

    def chunk_body(k, carry):
        e0 = base_e + k * CH
        pltpu.sync_copy(src_hbm.at[pl.ds(e0, CH)], src_v)
        for h in range(H):
            pltpu.sync_copy(sf_hbm.at[pl.ds(h * E + e0, CH)], sfc[h])
        for g in range(G):
            s16 = src_v[pl.ds(g * 16, 16)]
            for h in range(H):
                d0 = plsc.load_gather(dpart_v, [s16 + h * N])
                d1 = plsc.load_gather(dpart_v, [s16 + (H * N + h * N)])
                sfv = sfc[h][pl.ds(g * 16, 16)]
                alc[h][pl.ds(g * 16, 16)] = sfv / (d0 + d1)
        for h in range(H):
            pltpu.sync_copy(alc[h], alpha_hbm.at[pl.ds(h * E + e0, CH)])
        return carry

    lax.fori_loop(0, NCHUNK, chunk_body, 0)


def _sc2(src, sf, dpart_flat):
    mesh = plsc.VectorSubcoreMesh(core_axis_name="c", subcore_axis_name="s")
    return pl.kernel(
        _sc2_body,
        out_type=jax.ShapeDtypeStruct((H * E,), jnp.float32),
        mesh=mesh,
        compiler_params=pltpu.CompilerParams(needs_layout_passes=False),
        scratch_types=[
            pltpu.VMEM((NC * H * N,), jnp.float32),
            pltpu.VMEM((CH,), jnp.int32),
            [pltpu.VMEM((CH,), jnp.float32) for _ in range(H)],
            [pltpu.VMEM((CH,), jnp.float32) for _ in range(H)],
        ],
    )(src, sf, dpart_flat)


def _sc3_body(hidx, src_hbm, dst_hbm, alpha_hbm, fch_hbm, zerosf_hbm,
              agg_hbm,
              src_v, dst_v, af, frow_v, stage, agg_sp, sem):
    c = lax.axis_index("c")
    s = lax.axis_index("s")
    cwid = c * NS + s
    base_e = cwid * EB

    @pl.when(s == 0)
    def _():
        pltpu.sync_copy(zerosf_hbm, agg_sp)

    plsc.subcore_barrier()

    def chunk_body(k, carry):
        e0 = base_e + k * CH
        pltpu.sync_copy(src_hbm.at[pl.ds(e0, CH)], src_v)
        pltpu.sync_copy(dst_hbm.at[pl.ds(e0, CH)], dst_v)
        pltpu.sync_copy(alpha_hbm.at[pl.ds(hidx * E + e0, CH)], af)
        pltpu.async_copy(fch_hbm.at[dst_v], frow_v, sem).wait()

        def edge_body(e, ecarry):
            av = plsc.load_gather(af, [jnp.full((16,), e, dtype=jnp.int32)])
            for j in range(F // 16):
                stage[e, pl.ds(j * 16, 16)] = av * frow_v[e, pl.ds(j * 16, 16)]
            return ecarry

        lax.fori_loop(0, CH, edge_body, 0)
        pltpu.sync_copy(stage, agg_sp.at[src_v], add=True)
        return carry

    lax.fori_loop(0, NCHUNK, chunk_body, 0)

    plsc.subcore_barrier()

    @pl.when(s == 0)
    def _():
        pltpu.sync_copy(agg_sp, agg_hbm.at[c])


def _sc3(src, dst, alpha, fc_h, zerosf, hidx):
    mesh = plsc.VectorSubcoreMesh(core_axis_name="c", subcore_axis_name="s")
    return pl.kernel(
        functools.partial(_sc3_body, hidx),
        out_type=jax.ShapeDtypeStruct((NC, N, F), jnp.float32),
        mesh=mesh,
        compiler_params=pltpu.CompilerParams(needs_layout_passes=False),
        scratch_types=[
            pltpu.VMEM((CH,), jnp.int32),
            pltpu.VMEM((CH,), jnp.int32),
            pltpu.VMEM((CH,), jnp.float32),
            pltpu.VMEM((CH, F), jnp.float32),
            pltpu.VMEM((CH, F), jnp.float32),
            pltpu.VMEM_SHARED((N, F), jnp.float32),
            pltpu.SemaphoreType.DMA,
        ],
    )(src, dst, alpha, fc_h, zerosf)


def _tc2_body(*refs):
    head_refs = refs[:H]
    out_ref = refs[H]
    v = jnp.concatenate([r[:, 0, :] + r[:, 1, :] for r in head_refs], axis=1)
    out_ref[...] = jnp.where(v > 0, v, jnp.exp(v) - 1.0)


def _tc2(agg_heads):
    bn = 1000
    grid = (N // bn,)
    return pl.pallas_call(
        _tc2_body,
        grid=grid,
        in_specs=[pl.BlockSpec((bn, NC, F), lambda i: (i, 0, 0))
                  for _ in range(H)],
        out_specs=pl.BlockSpec((bn, H * F), lambda i: (i, 0)),
        out_shape=jax.ShapeDtypeStruct((N, H * F), jnp.float32),
    )(*agg_heads)


def kernel(x, edge_index, attn_w, attn_b, fc_w, fc_b):
    src = edge_index[0].astype(jnp.int32)
    dst = edge_index[1].astype(jnp.int32)

    w8 = jnp.concatenate([attn_w[:, :F].T, attn_w[:, F:].T], axis=1)
    b8 = jnp.concatenate([attn_b, jnp.zeros((H,), jnp.float32)])

    fc_h, snode = _tc1(x, fc_w, fc_b, w8, b8)

    zerosn = jnp.zeros((N,), jnp.float32)
    zerosf = jnp.zeros((N, F), jnp.float32)

    sf, dpart = _sc1(src, dst, snode.reshape(-1), zerosn)
    alpha = _sc2(src, sf, dpart.reshape(-1))

    agg_heads = []
    for hidx in range(H):
        part = _sc3(src, dst, alpha, fc_h, zerosf, hidx)
        agg_heads.append(part.transpose(1, 0, 2))

    return _tc2(agg_heads)


# speedup vs baseline: 13.6008x; 1.0027x over previous
import functools

import jax
import jax.numpy as jnp
from jax import lax
from jax.experimental import pallas as pl
from jax.experimental.pallas import tpu as pltpu
from jax.experimental.pallas import tpu_sc as plsc

N = 10000
E = 320000
F = 128
H = 4
FC = 32
NC = 2
NS = 16
NW = NC * NS
EB = E // NW
CH = 80
NCHUNK = EB // CH
G = CH // 16
CH3 = 40
NCH3 = EB // CH3


def _iota16():
    return lax.iota(jnp.int32, 16)


def _full16(v):
    return jnp.full((16,), v, dtype=jnp.int32)


def _lrelu(v):
    return jnp.where(v > 0, v, 0.2 * v)


def _tc1_body(x_ref, fcw_ref, fcb_ref, w8_ref, b8_ref, fch_ref, snode_ref):
    xb = x_ref[...]
    fch_ref[...] = (
        jnp.dot(xb, fcw_ref[...].T, preferred_element_type=jnp.float32)
        + fcb_ref[...][None, :]
    )
    snode_ref[...] = (
        jnp.dot(xb, w8_ref[...], preferred_element_type=jnp.float32)
        + b8_ref[...][None, :]
    )


def _tc1(x, fc_w, fc_b, w8, b8):
    bn = 1000
    grid = (N // bn,)
    return pl.pallas_call(
        _tc1_body,
        grid=grid,
        in_specs=[
            pl.BlockSpec((bn, F), lambda i: (i, 0)),
            pl.BlockSpec((F, F), lambda i: (0, 0)),
            pl.BlockSpec((F,), lambda i: (0,)),
            pl.BlockSpec((F, 2 * H), lambda i: (0, 0)),
            pl.BlockSpec((2 * H,), lambda i: (0,)),
        ],
        out_specs=[
            pl.BlockSpec((bn, F), lambda i: (i, 0)),
            pl.BlockSpec((bn, 2 * H), lambda i: (i, 0)),
        ],
        out_shape=[
            jax.ShapeDtypeStruct((N, F), jnp.float32),
            jax.ShapeDtypeStruct((N, 2 * H), jnp.float32),
        ],
    )(x, fc_w, fc_b, w8, b8)


def _sc1_body(src3_hbm, dst3_hbm, snode_hbm, zerosn_hbm,
              sf_hbm, dpart_hbm,
              snode_v, src_t, dst_t, sfb, srb, d_sp):
    c = lax.axis_index("c")
    s = lax.axis_index("s")
    cwid = c * NS + s
    base_e = cwid * EB

    pltpu.sync_copy(snode_hbm, snode_v)
    pltpu.sync_copy(src3_hbm.at[cwid], src_t)
    pltpu.sync_copy(dst3_hbm.at[cwid], dst_t)

    @pl.when(s == 0)
    def _():
        for h in range(H):
            pltpu.sync_copy(zerosn_hbm, d_sp[h])

    plsc.subcore_barrier()

    def chunk_body(k, carry):
        e0 = base_e + k * CH
        for g in range(G):
            sb = src_t[k, pl.ds(g * 16, 16)] * 8
            db = dst_t[k, pl.ds(g * 16, 16)] * 8
            for h in range(H):
                al_s = plsc.load_gather(snode_v, [sb + h])
                ar_s = plsc.load_gather(snode_v, [sb + (H + h)])
                al_d = plsc.load_gather(snode_v, [db + h])
                ar_d = plsc.load_gather(snode_v, [db + (H + h)])
                sfb[h][pl.ds(g * 16, 16)] = _lrelu(al_s + ar_d)
                srb[h][pl.ds(g * 16, 16)] = _lrelu(al_d + ar_s)
        for h in range(H):
            pltpu.sync_copy(sfb[h], sf_hbm.at[pl.ds(h * E + e0, CH)])
            pltpu.sync_copy(sfb[h], d_sp[h].at[src_t.at[k]], add=True)
            pltpu.sync_copy(srb[h], d_sp[h].at[dst_t.at[k]], add=True)
        return carry

    lax.fori_loop(0, NCHUNK, chunk_body, 0)

    plsc.subcore_barrier()

    @pl.when(s == 0)
    def _():
        for h in range(H):
            pltpu.sync_copy(d_sp[h], dpart_hbm.at[c, h])


def _sc1(src3, dst3, snode_flat, zerosn):
    mesh = plsc.VectorSubcoreMesh(core_axis_name="c", subcore_axis_name="s")
    return pl.kernel(
        _sc1_body,
        out_type=[
            jax.ShapeDtypeStruct((H * E,), jnp.float32),
            jax.ShapeDtypeStruct((NC, H, N), jnp.float32),
        ],
        mesh=mesh,
        compiler_params=pltpu.CompilerParams(needs_layout_passes=False),
        scratch_types=[
            pltpu.VMEM((N * 2 * H,), jnp.float32),
            pltpu.VMEM((NCHUNK, CH), jnp.int32),
            pltpu.VMEM((NCHUNK, CH), jnp.int32),
            [pltpu.VMEM((CH,), jnp.float32) for _ in range(H)],
            [pltpu.VMEM((CH,), jnp.float32) for _ in range(H)],
            [pltpu.VMEM_SHARED((N,), jnp.float32) for _ in range(H)],
        ],
    )(src3, dst3, snode_flat, zerosn)


def _sc2_body(src_hbm, sf_hbm, dpart_hbm,
              alpha_hbm,
              dpart_v, src_v, sfc, alc):
    c = lax.axis_index("c")
    s = lax.axis_index("s")
    cwid = c * NS + s
    base_e = cwid * EB

    pltpu.sync_copy(dpart_hbm, dpart_v)

    def chunk_body(k, carry):
        e0 = base_e + k * CH
        pltpu.sync_copy(src_hbm.at[pl.ds(e0, CH)], src_v)
        for h in range(H):
            pltpu.sync_copy(sf_hbm.at[pl.ds(h * E + e0, CH)], sfc[h])
        for g in range(G):
            s16 = src_v[pl.ds(g * 16, 16)]
            for h in range(H):
                d0 = plsc.load_gather(dpart_v, [s16 + h * N])
                d1 = plsc.load_gather(dpart_v, [s16 + (H * N + h * N)])
                sfv = sfc[h][pl.ds(g * 16, 16)]
                alc[h][pl.ds(g * 16, 16)] = sfv / (d0 + d1)
        for h in range(H):
            pltpu.sync_copy(alc[h], alpha_hbm.at[pl.ds(h * E + e0, CH)])
        return carry

    lax.fori_loop(0, NCHUNK, chunk_body, 0)


def _sc2(src, sf, dpart_flat):
    mesh = plsc.VectorSubcoreMesh(core_axis_name="c", subcore_axis_name="s")
    return pl.kernel(
        _sc2_body,
        out_type=jax.ShapeDtypeStruct((H * E,), jnp.float32),
        mesh=mesh,
        compiler_params=pltpu.CompilerParams(needs_layout_passes=False),
        scratch_types=[
            pltpu.VMEM((NC * H * N,), jnp.float32),
            pltpu.VMEM((CH,), jnp.int32),
            [pltpu.VMEM((CH,), jnp.float32) for _ in range(H)],
            [pltpu.VMEM((CH,), jnp.float32) for _ in range(H)],
        ],
    )(src, sf, dpart_flat)


def _sc3_body(hidx, src3_hbm, dst3_hbm, alpha_hbm, fch_hbm,
              agg_hbm,
              src_c, dst_c, af_c, frows, stages, agg_sp, gsems, ssems):
    c = lax.axis_index("c")
    s = lax.axis_index("s")
    cwid = c * NS + s
    base_e = cwid * EB

    def zb(e, zcarry):
        for j in range(F // 16):
            stages[0][e, pl.ds(j * 16, 16)] = jnp.zeros((16,), jnp.float32)
        return zcarry

    lax.fori_loop(0, CH3, zb, 0)

    row0 = s * 624
    for t in range(15):
        pltpu.sync_copy(stages[0], agg_sp.at[pl.ds(row0 + t * CH3, CH3)])
    pltpu.sync_copy(stages[0].at[pl.ds(0, 24)], agg_sp.at[pl.ds(row0 + 600, 24)])

    @pl.when(s == NS - 1)
    def _():
        pltpu.sync_copy(stages[0].at[pl.ds(0, 16)], agg_sp.at[pl.ds(9984, 16)])

    plsc.subcore_barrier()

    def load_chunk(q, b):
        pltpu.sync_copy(src3_hbm.at[cwid, pl.ds(q, 1)], src_c[b])
        pltpu.sync_copy(dst3_hbm.at[cwid, pl.ds(q, 1)], dst_c[b])
        pltpu.sync_copy(alpha_hbm.at[pl.ds(hidx * E + base_e + q * CH3, CH3)],
                        af_c[b])
        pltpu.async_copy(fch_hbm.at[dst_c[b].at[0]], frows[b], gsems[b])

    def wait_gather(b):
        pltpu.make_async_copy(fch_hbm.at[dst_c[b].at[0]], frows[b],
                              gsems[b]).wait()

    def compute(b):
        frow = frows[b]
        stage = stages[b]
        af = af_c[b]

        def edge_body(i, ecarry):
            for u in range(2):
                e = i * 2 + u
                av = plsc.load_gather(af, [jnp.full((16,), e, dtype=jnp.int32)])
                for j in range(F // 16):
                    stage[e, pl.ds(j * 16, 16)] = av * frow[e, pl.ds(j * 16, 16)]
            return ecarry

        lax.fori_loop(0, CH3 // 2, edge_body, 0)

    def start_scatter(b):
        return pltpu.async_copy(stages[b], agg_sp.at[src_c[b].at[0]], ssems[b],
                                add=True)

    load_chunk(0, 0)

    def pair_body(i, carry):
        qb = 2 * i + 1
        qn = jnp.minimum(2 * i + 2, NCH3 - 1)
        load_chunk(qb, 1)
        wait_gather(0)
        compute(0)
        da = start_scatter(0)
        load_chunk(qn, 0)
        da.wait()
        wait_gather(1)
        compute(1)
        start_scatter(1).wait()
        return carry

    lax.fori_loop(0, NCH3 // 2, pair_body, 0)

    wait_gather(0)

    plsc.subcore_barrier()

    for t in range(15):
        pltpu.sync_copy(agg_sp.at[pl.ds(row0 + t * CH3, CH3)],
                        agg_hbm.at[c, pl.ds(row0 + t * CH3, CH3)])
    pltpu.sync_copy(agg_sp.at[pl.ds(row0 + 600, 24)],
                    agg_hbm.at[c, pl.ds(row0 + 600, 24)])

    @pl.when(s == NS - 1)
    def _():
        pltpu.sync_copy(agg_sp.at[pl.ds(9984, 16)],
                        agg_hbm.at[c, pl.ds(9984, 16)])


def _sc3(src3b, dst3b, alpha, fc_h, hidx):
    mesh = plsc.VectorSubcoreMesh(core_axis_name="c", subcore_axis_name="s")
    return pl.kernel(
        functools.partial(_sc3_body, hidx),
        out_type=jax.ShapeDtypeStruct((NC, N, F), jnp.float32),
        mesh=mesh,
        compiler_params=pltpu.CompilerParams(needs_layout_passes=False),
        scratch_types=[
            [pltpu.VMEM((1, CH3), jnp.int32) for _ in range(2)],
            [pltpu.VMEM((1, CH3), jnp.int32) for _ in range(2)],
            [pltpu.VMEM((CH3,), jnp.float32) for _ in range(2)],
            [pltpu.VMEM((CH3, F), jnp.float32) for _ in range(2)],
            [pltpu.VMEM((CH3, F), jnp.float32) for _ in range(2)],
            pltpu.VMEM_SHARED((N, F), jnp.float32),
            [pltpu.SemaphoreType.DMA for _ in range(2)],
            [pltpu.SemaphoreType.DMA for _ in range(2)],
        ],
    )(src3b, dst3b, alpha, fc_h)


def _tc2_body(*refs):
    head_refs = refs[:H]
    out_ref = refs[H]
    v = jnp.concatenate([r[:, 0, :] + r[:, 1, :] for r in head_refs], axis=1)
    out_ref[...] = jnp.where(v > 0, v, jnp.exp(v) - 1.0)


def _tc2(agg_heads):
    bn = 1000
    grid = (N // bn,)
    return pl.pallas_call(
        _tc2_body,
        grid=grid,
        in_specs=[pl.BlockSpec((bn, NC, F), lambda i: (i, 0, 0))
                  for _ in range(H)],
        out_specs=pl.BlockSpec((bn, H * F), lambda i: (i, 0)),
        out_shape=jax.ShapeDtypeStruct((N, H * F), jnp.float32),
    )(*agg_heads)


def kernel(x, edge_index, attn_w, attn_b, fc_w, fc_b):
    src = edge_index[0].astype(jnp.int32)
    dst = edge_index[1].astype(jnp.int32)

    w8 = jnp.concatenate([attn_w[:, :F].T, attn_w[:, F:].T], axis=1)
    b8 = jnp.concatenate([attn_b, jnp.zeros((H,), jnp.float32)])

    fc_h, snode = _tc1(x, fc_w, fc_b, w8, b8)

    zerosn = jnp.zeros((N,), jnp.float32)

    src3 = src.reshape(NW, NCHUNK, CH)
    dst3 = dst.reshape(NW, NCHUNK, CH)
    src3b = src.reshape(NW, NCH3, CH3)
    dst3b = dst.reshape(NW, NCH3, CH3)

    sf, dpart = _sc1(src3, dst3, snode.reshape(-1), zerosn)
    alpha = _sc2(src, sf, dpart.reshape(-1))

    agg_heads = []
    for hidx in range(H):
        part = _sc3(src3b, dst3b, alpha, fc_h, hidx)
        agg_heads.append(part.transpose(1, 0, 2))

    return _tc2(agg_heads)


# speedup vs baseline: 20.4811x; 1.5059x over previous
import functools

import jax
import jax.numpy as jnp
from jax import lax
from jax.experimental import pallas as pl
from jax.experimental.pallas import tpu as pltpu
from jax.experimental.pallas import tpu_sc as plsc

N = 10000
E = 320000
F = 128
H = 4
FC = 32
NC = 2
NS = 16
NW = NC * NS
EB = E // NW
CH = 80
NCHUNK = EB // CH
G = CH // 16
CH3 = 40
NCH3 = EB // CH3


def _iota16():
    return lax.iota(jnp.int32, 16)


def _full16(v):
    return jnp.full((16,), v, dtype=jnp.int32)


def _lrelu(v):
    return jnp.where(v > 0, v, 0.2 * v)


def _tc1_body(x_ref, fcw_ref, fcb_ref, w8_ref, b8_ref, fch_ref, snode_ref):
    xb = x_ref[...]
    fch_ref[...] = (
        jnp.dot(xb, fcw_ref[...].T, preferred_element_type=jnp.float32)
        + fcb_ref[...][None, :]
    )
    snode_ref[...] = (
        jnp.dot(xb, w8_ref[...], preferred_element_type=jnp.float32)
        + b8_ref[...][None, :]
    )


def _tc1(x, fc_w, fc_b, w8, b8):
    bn = 1000
    grid = (N // bn,)
    return pl.pallas_call(
        _tc1_body,
        grid=grid,
        in_specs=[
            pl.BlockSpec((bn, F), lambda i: (i, 0)),
            pl.BlockSpec((F, F), lambda i: (0, 0)),
            pl.BlockSpec((F,), lambda i: (0,)),
            pl.BlockSpec((F, 2 * H), lambda i: (0, 0)),
            pl.BlockSpec((2 * H,), lambda i: (0,)),
        ],
        out_specs=[
            pl.BlockSpec((bn, F), lambda i: (i, 0)),
            pl.BlockSpec((bn, 2 * H), lambda i: (i, 0)),
        ],
        out_shape=[
            jax.ShapeDtypeStruct((N, F), jnp.float32),
            jax.ShapeDtypeStruct((N, 2 * H), jnp.float32),
        ],
    )(x, fc_w, fc_b, w8, b8)


def _sc1_body(src3_hbm, dst3_hbm, snode_hbm, zerosn_hbm,
              sf_hbm, dpart_hbm,
              snode_v, src_t, dst_t, sfb, srb, d_sp):
    c = lax.axis_index("c")
    s = lax.axis_index("s")
    cwid = c * NS + s
    base_e = cwid * EB

    pltpu.sync_copy(snode_hbm, snode_v)
    pltpu.sync_copy(src3_hbm.at[cwid], src_t)
    pltpu.sync_copy(dst3_hbm.at[cwid], dst_t)

    @pl.when(s == 0)
    def _():
        for h in range(H):
            pltpu.sync_copy(zerosn_hbm, d_sp[h])

    plsc.subcore_barrier()

    def chunk_body(k, carry):
        e0 = base_e + k * CH
        for g in range(G):
            sb = src_t[k, pl.ds(g * 16, 16)] * 8
            db = dst_t[k, pl.ds(g * 16, 16)] * 8
            for h in range(H):
                al_s = plsc.load_gather(snode_v, [sb + h])
                ar_s = plsc.load_gather(snode_v, [sb + (H + h)])
                al_d = plsc.load_gather(snode_v, [db + h])
                ar_d = plsc.load_gather(snode_v, [db + (H + h)])
                sfb[h][pl.ds(g * 16, 16)] = _lrelu(al_s + ar_d)
                srb[h][pl.ds(g * 16, 16)] = _lrelu(al_d + ar_s)
        for h in range(H):
            pltpu.sync_copy(sfb[h], sf_hbm.at[pl.ds(h * E + e0, CH)])
            pltpu.sync_copy(sfb[h], d_sp[h].at[src_t.at[k]], add=True)
            pltpu.sync_copy(srb[h], d_sp[h].at[dst_t.at[k]], add=True)
        return carry

    lax.fori_loop(0, NCHUNK, chunk_body, 0)

    plsc.subcore_barrier()

    @pl.when(s == 0)
    def _():
        for h in range(H):
            pltpu.sync_copy(d_sp[h], dpart_hbm.at[c, h])


def _sc1(src3, dst3, snode_flat, zerosn):
    mesh = plsc.VectorSubcoreMesh(core_axis_name="c", subcore_axis_name="s")
    return pl.kernel(
        _sc1_body,
        out_type=[
            jax.ShapeDtypeStruct((H * E,), jnp.float32),
            jax.ShapeDtypeStruct((NC, H, N), jnp.float32),
        ],
        mesh=mesh,
        compiler_params=pltpu.CompilerParams(needs_layout_passes=False),
        scratch_types=[
            pltpu.VMEM((N * 2 * H,), jnp.float32),
            pltpu.VMEM((NCHUNK, CH), jnp.int32),
            pltpu.VMEM((NCHUNK, CH), jnp.int32),
            [pltpu.VMEM((CH,), jnp.float32) for _ in range(H)],
            [pltpu.VMEM((CH,), jnp.float32) for _ in range(H)],
            [pltpu.VMEM_SHARED((N,), jnp.float32) for _ in range(H)],
        ],
    )(src3, dst3, snode_flat, zerosn)


def _sc2_body(src_hbm, sf_hbm, dpart_hbm,
              alpha_hbm,
              dpart_v, src_v, sfc, alc):
    c = lax.axis_index("c")
    s = lax.axis_index("s")
    cwid = c * NS + s
    base_e = cwid * EB

    pltpu.sync_copy(dpart_hbm, dpart_v)

    def chunk_body(k, carry):
        e0 = base_e + k * CH
        pltpu.sync_copy(src_hbm.at[pl.ds(e0, CH)], src_v)
        for h in range(H):
            pltpu.sync_copy(sf_hbm.at[pl.ds(h * E + e0, CH)], sfc[h])
        for g in range(G):
            s16 = src_v[pl.ds(g * 16, 16)]
            for h in range(H):
                d0 = plsc.load_gather(dpart_v, [s16 + h * N])
                d1 = plsc.load_gather(dpart_v, [s16 + (H * N + h * N)])
                sfv = sfc[h][pl.ds(g * 16, 16)]
                alc[h][pl.ds(g * 16, 16)] = sfv / (d0 + d1)
        for h in range(H):
            pltpu.sync_copy(alc[h], alpha_hbm.at[pl.ds(h * E + e0, CH)])
        return carry

    lax.fori_loop(0, NCHUNK, chunk_body, 0)


def _sc2(src, sf, dpart_flat):
    mesh = plsc.VectorSubcoreMesh(core_axis_name="c", subcore_axis_name="s")
    return pl.kernel(
        _sc2_body,
        out_type=jax.ShapeDtypeStruct((H * E,), jnp.float32),
        mesh=mesh,
        compiler_params=pltpu.CompilerParams(needs_layout_passes=False),
        scratch_types=[
            pltpu.VMEM((NC * H * N,), jnp.float32),
            pltpu.VMEM((CH,), jnp.int32),
            [pltpu.VMEM((CH,), jnp.float32) for _ in range(H)],
            [pltpu.VMEM((CH,), jnp.float32) for _ in range(H)],
        ],
    )(src, sf, dpart_flat)


def _sc3_body(hidx, src3_hbm, dst3_hbm, alpha_hbm, fch_hbm,
              agg_hbm,
              src_c, dst_c, af_c, frows, stages, agg_sp, gsems, ssems):
    c = lax.axis_index("c")
    s = lax.axis_index("s")
    cwid = c * NS + s
    base_e = cwid * EB

    def zb(e, zcarry):
        for j in range(F // 16):
            stages[0][e, pl.ds(j * 16, 16)] = jnp.zeros((16,), jnp.float32)
        return zcarry

    lax.fori_loop(0, CH3, zb, 0)

    row0 = s * 624
    for t in range(15):
        pltpu.sync_copy(stages[0], agg_sp.at[pl.ds(row0 + t * CH3, CH3)])
    pltpu.sync_copy(stages[0].at[pl.ds(0, 24)], agg_sp.at[pl.ds(row0 + 600, 24)])

    @pl.when(s == NS - 1)
    def _():
        pltpu.sync_copy(stages[0].at[pl.ds(0, 16)], agg_sp.at[pl.ds(9984, 16)])

    plsc.subcore_barrier()

    def load_chunk(q, b):
        pltpu.sync_copy(src3_hbm.at[cwid, pl.ds(q, 1)], src_c[b])
        pltpu.sync_copy(dst3_hbm.at[cwid, pl.ds(q, 1)], dst_c[b])
        pltpu.sync_copy(alpha_hbm.at[pl.ds(hidx * E + base_e + q * CH3, CH3)],
                        af_c[b])
        pltpu.async_copy(fch_hbm.at[dst_c[b].at[0]], frows[b], gsems[b])

    def wait_gather(b):
        pltpu.make_async_copy(fch_hbm.at[dst_c[b].at[0]], frows[b],
                              gsems[b]).wait()

    def compute(b):
        frow = frows[b]
        stage = stages[b]
        af = af_c[b]

        @plsc.parallel_loop(0, CH3, step=1, unroll=4)
        def edge_body(e):
            av = plsc.load_gather(af, [jnp.full((16,), e, dtype=jnp.int32)])
            for j in range(F // 16):
                stage[e, pl.ds(j * 16, 16)] = av * frow[e, pl.ds(j * 16, 16)]

    def start_scatter(b):
        return pltpu.async_copy(stages[b], agg_sp.at[src_c[b].at[0]], ssems[b],
                                add=True)

    load_chunk(0, 0)

    def pair_body(i, carry):
        qb = 2 * i + 1
        qn = jnp.minimum(2 * i + 2, NCH3 - 1)
        load_chunk(qb, 1)
        wait_gather(0)
        compute(0)
        da = start_scatter(0)
        load_chunk(qn, 0)
        da.wait()
        wait_gather(1)
        compute(1)
        start_scatter(1).wait()
        return carry

    lax.fori_loop(0, NCH3 // 2, pair_body, 0)

    wait_gather(0)

    plsc.subcore_barrier()

    for t in range(15):
        pltpu.sync_copy(agg_sp.at[pl.ds(row0 + t * CH3, CH3)],
                        agg_hbm.at[c, pl.ds(row0 + t * CH3, CH3)])
    pltpu.sync_copy(agg_sp.at[pl.ds(row0 + 600, 24)],
                    agg_hbm.at[c, pl.ds(row0 + 600, 24)])

    @pl.when(s == NS - 1)
    def _():
        pltpu.sync_copy(agg_sp.at[pl.ds(9984, 16)],
                        agg_hbm.at[c, pl.ds(9984, 16)])


def _sc3(src3b, dst3b, alpha, fc_h, hidx):
    mesh = plsc.VectorSubcoreMesh(core_axis_name="c", subcore_axis_name="s")
    return pl.kernel(
        functools.partial(_sc3_body, hidx),
        out_type=jax.ShapeDtypeStruct((NC, N, F), jnp.float32),
        mesh=mesh,
        compiler_params=pltpu.CompilerParams(needs_layout_passes=False),
        scratch_types=[
            [pltpu.VMEM((1, CH3), jnp.int32) for _ in range(2)],
            [pltpu.VMEM((1, CH3), jnp.int32) for _ in range(2)],
            [pltpu.VMEM((CH3,), jnp.float32) for _ in range(2)],
            [pltpu.VMEM((CH3, F), jnp.float32) for _ in range(2)],
            [pltpu.VMEM((CH3, F), jnp.float32) for _ in range(2)],
            pltpu.VMEM_SHARED((N, F), jnp.float32),
            [pltpu.SemaphoreType.DMA for _ in range(2)],
            [pltpu.SemaphoreType.DMA for _ in range(2)],
        ],
    )(src3b, dst3b, alpha, fc_h)


def _tc2_body(*refs):
    head_refs = refs[:H]
    out_ref = refs[H]
    v = jnp.concatenate([r[:, 0, :] + r[:, 1, :] for r in head_refs], axis=1)
    out_ref[...] = jnp.where(v > 0, v, jnp.exp(v) - 1.0)


def _tc2(agg_heads):
    bn = 1000
    grid = (N // bn,)
    return pl.pallas_call(
        _tc2_body,
        grid=grid,
        in_specs=[pl.BlockSpec((bn, NC, F), lambda i: (i, 0, 0))
                  for _ in range(H)],
        out_specs=pl.BlockSpec((bn, H * F), lambda i: (i, 0)),
        out_shape=jax.ShapeDtypeStruct((N, H * F), jnp.float32),
    )(*agg_heads)


def kernel(x, edge_index, attn_w, attn_b, fc_w, fc_b):
    src = edge_index[0].astype(jnp.int32)
    dst = edge_index[1].astype(jnp.int32)

    w8 = jnp.concatenate([attn_w[:, :F].T, attn_w[:, F:].T], axis=1)
    b8 = jnp.concatenate([attn_b, jnp.zeros((H,), jnp.float32)])

    fc_h, snode = _tc1(x, fc_w, fc_b, w8, b8)

    zerosn = jnp.zeros((N,), jnp.float32)

    src3 = src.reshape(NW, NCHUNK, CH)
    dst3 = dst.reshape(NW, NCHUNK, CH)
    src3b = src.reshape(NW, NCH3, CH3)
    dst3b = dst.reshape(NW, NCH3, CH3)

    sf, dpart = _sc1(src3, dst3, snode.reshape(-1), zerosn)
    alpha = _sc2(src, sf, dpart.reshape(-1))

    agg_heads = []
    for hidx in range(H):
        part = _sc3(src3b, dst3b, alpha, fc_h, hidx)
        agg_heads.append(part.transpose(1, 0, 2))

    return _tc2(agg_heads)


# speedup vs baseline: 29.1130x; 1.4215x over previous
import functools

import jax
import jax.numpy as jnp
from jax import lax
from jax.experimental import pallas as pl
from jax.experimental.pallas import tpu as pltpu
from jax.experimental.pallas import tpu_sc as plsc

N = 10000
E = 320000
F = 128
H = 4
FC = 32
NC = 2
NS = 16
NW = NC * NS
EB = E // NW
CH = 80
NCHUNK = EB // CH
G = CH // 16
CH3 = 80
NCH3 = EB // CH3


def _iota16():
    return lax.iota(jnp.int32, 16)


def _full16(v):
    return jnp.full((16,), v, dtype=jnp.int32)


def _lrelu(v):
    return jnp.where(v > 0, v, 0.2 * v)


def _tc1_body(x_ref, fcw_ref, fcb_ref, w8_ref, b8_ref, fch_ref, snode_ref):
    xb = x_ref[...]
    fch_ref[...] = (
        jnp.dot(xb, fcw_ref[...].T, preferred_element_type=jnp.float32)
        + fcb_ref[...][None, :]
    )
    snode_ref[...] = (
        jnp.dot(xb, w8_ref[...], preferred_element_type=jnp.float32)
        + b8_ref[...][None, :]
    )


def _tc1(x, fc_w, fc_b, w8, b8):
    bn = 1000
    grid = (N // bn,)
    return pl.pallas_call(
        _tc1_body,
        grid=grid,
        in_specs=[
            pl.BlockSpec((bn, F), lambda i: (i, 0)),
            pl.BlockSpec((F, F), lambda i: (0, 0)),
            pl.BlockSpec((F,), lambda i: (0,)),
            pl.BlockSpec((F, 2 * H), lambda i: (0, 0)),
            pl.BlockSpec((2 * H,), lambda i: (0,)),
        ],
        out_specs=[
            pl.BlockSpec((bn, F), lambda i: (i, 0)),
            pl.BlockSpec((bn, 2 * H), lambda i: (i, 0)),
        ],
        out_shape=[
            jax.ShapeDtypeStruct((N, F), jnp.float32),
            jax.ShapeDtypeStruct((N, 2 * H), jnp.float32),
        ],
    )(x, fc_w, fc_b, w8, b8)


def _sc1_body(src3_hbm, dst3_hbm, snode_hbm, zerosn_hbm,
              sf_hbm, dpart_hbm,
              snode_v, src_t, dst_t, sfb, srb, d_sp):
    c = lax.axis_index("c")
    s = lax.axis_index("s")
    cwid = c * NS + s
    base_e = cwid * EB

    pltpu.sync_copy(snode_hbm, snode_v)
    pltpu.sync_copy(src3_hbm.at[cwid], src_t)
    pltpu.sync_copy(dst3_hbm.at[cwid], dst_t)

    @pl.when(s == 0)
    def _():
        for h in range(H):
            pltpu.sync_copy(zerosn_hbm, d_sp[h])

    plsc.subcore_barrier()

    def chunk_body(k, carry):
        e0 = base_e + k * CH
        for g in range(G):
            sb = src_t[k, pl.ds(g * 16, 16)] * 8
            db = dst_t[k, pl.ds(g * 16, 16)] * 8
            for h in range(H):
                al_s = plsc.load_gather(snode_v, [sb + h])
                ar_s = plsc.load_gather(snode_v, [sb + (H + h)])
                al_d = plsc.load_gather(snode_v, [db + h])
                ar_d = plsc.load_gather(snode_v, [db + (H + h)])
                sfb[h][pl.ds(g * 16, 16)] = _lrelu(al_s + ar_d)
                srb[h][pl.ds(g * 16, 16)] = _lrelu(al_d + ar_s)
        for h in range(H):
            pltpu.sync_copy(sfb[h], sf_hbm.at[pl.ds(h * E + e0, CH)])
            pltpu.sync_copy(sfb[h], d_sp[h].at[src_t.at[k]], add=True)
            pltpu.sync_copy(srb[h], d_sp[h].at[dst_t.at[k]], add=True)
        return carry

    lax.fori_loop(0, NCHUNK, chunk_body, 0)

    plsc.subcore_barrier()

    @pl.when(s == 0)
    def _():
        for h in range(H):
            pltpu.sync_copy(d_sp[h], dpart_hbm.at[c, h])


def _sc1(src3, dst3, snode_flat, zerosn):
    mesh = plsc.VectorSubcoreMesh(core_axis_name="c", subcore_axis_name="s")
    return pl.kernel(
        _sc1_body,
        out_type=[
            jax.ShapeDtypeStruct((H * E,), jnp.float32),
            jax.ShapeDtypeStruct((NC, H, N), jnp.float32),
        ],
        mesh=mesh,
        compiler_params=pltpu.CompilerParams(needs_layout_passes=False),
        scratch_types=[
            pltpu.VMEM((N * 2 * H,), jnp.float32),
            pltpu.VMEM((NCHUNK, CH), jnp.int32),
            pltpu.VMEM((NCHUNK, CH), jnp.int32),
            [pltpu.VMEM((CH,), jnp.float32) for _ in range(H)],
            [pltpu.VMEM((CH,), jnp.float32) for _ in range(H)],
            [pltpu.VMEM_SHARED((N,), jnp.float32) for _ in range(H)],
        ],
    )(src3, dst3, snode_flat, zerosn)


def _sc2_body(src_hbm, sf_hbm, dpart_hbm,
              alpha_hbm,
              dpart_v, src_v, sfc, alc):
    c = lax.axis_index("c")
    s = lax.axis_index("s")
    cwid = c * NS + s
    base_e = cwid * EB

    pltpu.sync_copy(dpart_hbm, dpart_v)

    def chunk_body(k, carry):
        e0 = base_e + k * CH
        pltpu.sync_copy(src_hbm.at[pl.ds(e0, CH)], src_v)
        for h in range(H):
            pltpu.sync_copy(sf_hbm.at[pl.ds(h * E + e0, CH)], sfc[h])
        for g in range(G):
            s16 = src_v[pl.ds(g * 16, 16)]
            for h in range(H):
                d0 = plsc.load_gather(dpart_v, [s16 + h * N])
                d1 = plsc.load_gather(dpart_v, [s16 + (H * N + h * N)])
                sfv = sfc[h][pl.ds(g * 16, 16)]
                alc[h][pl.ds(g * 16, 16)] = sfv / (d0 + d1)
        for h in range(H):
            pltpu.sync_copy(alc[h], alpha_hbm.at[pl.ds(h * E + e0, CH)])
        return carry

    lax.fori_loop(0, NCHUNK, chunk_body, 0)


def _sc2(src, sf, dpart_flat):
    mesh = plsc.VectorSubcoreMesh(core_axis_name="c", subcore_axis_name="s")
    return pl.kernel(
        _sc2_body,
        out_type=jax.ShapeDtypeStruct((H * E,), jnp.float32),
        mesh=mesh,
        compiler_params=pltpu.CompilerParams(needs_layout_passes=False),
        scratch_types=[
            pltpu.VMEM((NC * H * N,), jnp.float32),
            pltpu.VMEM((CH,), jnp.int32),
            [pltpu.VMEM((CH,), jnp.float32) for _ in range(H)],
            [pltpu.VMEM((CH,), jnp.float32) for _ in range(H)],
        ],
    )(src, sf, dpart_flat)


def _sc3_body(hidx, src3_hbm, dst3_hbm, alpha_hbm, fch_hbm,
              agg_hbm,
              src_c, dst_c, af_c, frows, stages, agg_sp, gsems, ssems):
    c = lax.axis_index("c")
    s = lax.axis_index("s")
    cwid = c * NS + s
    base_e = cwid * EB

    def zb(e, zcarry):
        for j in range(F // 16):
            stages[0][e, pl.ds(j * 16, 16)] = jnp.zeros((16,), jnp.float32)
        return zcarry

    lax.fori_loop(0, CH3, zb, 0)

    row0 = s * 624
    for t in range(7):
        pltpu.sync_copy(stages[0], agg_sp.at[pl.ds(row0 + t * CH3, CH3)])
    pltpu.sync_copy(stages[0].at[pl.ds(0, 64)], agg_sp.at[pl.ds(row0 + 560, 64)])

    @pl.when(s == NS - 1)
    def _():
        pltpu.sync_copy(stages[0].at[pl.ds(0, 16)], agg_sp.at[pl.ds(9984, 16)])

    plsc.subcore_barrier()

    def load_chunk(q, b):
        pltpu.sync_copy(src3_hbm.at[cwid, pl.ds(q, 1)], src_c[b])
        pltpu.sync_copy(dst3_hbm.at[cwid, pl.ds(q, 1)], dst_c[b])
        pltpu.sync_copy(alpha_hbm.at[pl.ds(hidx * E + base_e + q * CH3, CH3)],
                        af_c[b])
        pltpu.async_copy(fch_hbm.at[dst_c[b].at[0]], frows[b], gsems[b])

    def wait_gather(b):
        pltpu.make_async_copy(fch_hbm.at[dst_c[b].at[0]], frows[b],
                              gsems[b]).wait()

    def compute(b):
        frow = frows[b]
        stage = stages[b]
        af = af_c[b]

        @plsc.parallel_loop(0, CH3, step=1, unroll=8)
        def edge_body(e):
            av = plsc.load_gather(af, [jnp.full((16,), e, dtype=jnp.int32)])
            for j in range(F // 16):
                stage[e, pl.ds(j * 16, 16)] = av * frow[e, pl.ds(j * 16, 16)]

    def start_scatter(b):
        return pltpu.async_copy(stages[b], agg_sp.at[src_c[b].at[0]], ssems[b],
                                add=True)

    load_chunk(0, 0)

    def pair_body(i, carry):
        qb = 2 * i + 1
        qn = jnp.minimum(2 * i + 2, NCH3 - 1)
        load_chunk(qb, 1)
        wait_gather(0)
        compute(0)
        da = start_scatter(0)
        load_chunk(qn, 0)
        da.wait()
        wait_gather(1)
        compute(1)
        start_scatter(1).wait()
        return carry

    lax.fori_loop(0, (NCH3 - 1) // 2, pair_body, 0)

    wait_gather(0)
    compute(0)
    start_scatter(0).wait()

    plsc.subcore_barrier()

    for t in range(7):
        pltpu.sync_copy(agg_sp.at[pl.ds(row0 + t * CH3, CH3)],
                        agg_hbm.at[c, pl.ds(row0 + t * CH3, CH3)])
    pltpu.sync_copy(agg_sp.at[pl.ds(row0 + 560, 64)],
                    agg_hbm.at[c, pl.ds(row0 + 560, 64)])

    @pl.when(s == NS - 1)
    def _():
        pltpu.sync_copy(agg_sp.at[pl.ds(9984, 16)],
                        agg_hbm.at[c, pl.ds(9984, 16)])


def _sc3(src3b, dst3b, alpha, fc_h, hidx):
    mesh = plsc.VectorSubcoreMesh(core_axis_name="c", subcore_axis_name="s")
    return pl.kernel(
        functools.partial(_sc3_body, hidx),
        out_type=jax.ShapeDtypeStruct((NC, N, F), jnp.float32),
        mesh=mesh,
        compiler_params=pltpu.CompilerParams(needs_layout_passes=False),
        scratch_types=[
            [pltpu.VMEM((1, CH3), jnp.int32) for _ in range(2)],
            [pltpu.VMEM((1, CH3), jnp.int32) for _ in range(2)],
            [pltpu.VMEM((CH3,), jnp.float32) for _ in range(2)],
            [pltpu.VMEM((CH3, F), jnp.float32) for _ in range(2)],
            [pltpu.VMEM((CH3, F), jnp.float32) for _ in range(2)],
            pltpu.VMEM_SHARED((N, F), jnp.float32),
            [pltpu.SemaphoreType.DMA for _ in range(2)],
            [pltpu.SemaphoreType.DMA for _ in range(2)],
        ],
    )(src3b, dst3b, alpha, fc_h)


def _tc2_body(*refs):
    head_refs = refs[:H]
    out_ref = refs[H]
    v = jnp.concatenate([r[:, 0, :] + r[:, 1, :] for r in head_refs], axis=1)
    out_ref[...] = jnp.where(v > 0, v, jnp.exp(v) - 1.0)


def _tc2(agg_heads):
    bn = 1000
    grid = (N // bn,)
    return pl.pallas_call(
        _tc2_body,
        grid=grid,
        in_specs=[pl.BlockSpec((bn, NC, F), lambda i: (i, 0, 0))
                  for _ in range(H)],
        out_specs=pl.BlockSpec((bn, H * F), lambda i: (i, 0)),
        out_shape=jax.ShapeDtypeStruct((N, H * F), jnp.float32),
    )(*agg_heads)


def kernel(x, edge_index, attn_w, attn_b, fc_w, fc_b):
    src = edge_index[0].astype(jnp.int32)
    dst = edge_index[1].astype(jnp.int32)

    w8 = jnp.concatenate([attn_w[:, :F].T, attn_w[:, F:].T], axis=1)
    b8 = jnp.concatenate([attn_b, jnp.zeros((H,), jnp.float32)])

    fc_h, snode = _tc1(x, fc_w, fc_b, w8, b8)

    zerosn = jnp.zeros((N,), jnp.float32)

    src3 = src.reshape(NW, NCHUNK, CH)
    dst3 = dst.reshape(NW, NCHUNK, CH)
    src3b = src.reshape(NW, NCH3, CH3)
    dst3b = dst.reshape(NW, NCH3, CH3)

    sf, dpart = _sc1(src3, dst3, snode.reshape(-1), zerosn)
    alpha = _sc2(src, sf, dpart.reshape(-1))

    agg_heads = []
    for hidx in range(H):
        part = _sc3(src3b, dst3b, alpha, fc_h, hidx)
        agg_heads.append(part.transpose(1, 0, 2))

    return _tc2(agg_heads)


# speedup vs baseline: 29.9008x; 1.0271x over previous
import functools

import jax
import jax.numpy as jnp
from jax import lax
from jax.experimental import pallas as pl
from jax.experimental.pallas import tpu as pltpu
from jax.experimental.pallas import tpu_sc as plsc

N = 10000
E = 320000
F = 128
H = 4
FC = 32
NC = 2
NS = 16
NW = NC * NS
EB = E // NW
CH = 80
NCHUNK = EB // CH
G = CH // 16
CH3 = 80
NCH3 = EB // CH3


def _iota16():
    return lax.iota(jnp.int32, 16)


def _full16(v):
    return jnp.full((16,), v, dtype=jnp.int32)


def _lrelu(v):
    return jnp.where(v > 0, v, 0.2 * v)


def _tc1_body(x_ref, fcw_ref, fcb_ref, w8_ref, b8_ref, fch_ref, snode_ref):
    xb = x_ref[...]
    fch_ref[...] = (
        jnp.dot(xb, fcw_ref[...].T, preferred_element_type=jnp.float32)
        + fcb_ref[...][None, :]
    )
    snode_ref[...] = (
        jnp.dot(xb, w8_ref[...], preferred_element_type=jnp.float32)
        + b8_ref[...][None, :]
    )


def _tc1(x, fc_w, fc_b, w8, b8):
    bn = 1000
    grid = (N // bn,)
    return pl.pallas_call(
        _tc1_body,
        grid=grid,
        in_specs=[
            pl.BlockSpec((bn, F), lambda i: (i, 0)),
            pl.BlockSpec((F, F), lambda i: (0, 0)),
            pl.BlockSpec((F,), lambda i: (0,)),
            pl.BlockSpec((F, 2 * H), lambda i: (0, 0)),
            pl.BlockSpec((2 * H,), lambda i: (0,)),
        ],
        out_specs=[
            pl.BlockSpec((bn, F), lambda i: (i, 0)),
            pl.BlockSpec((bn, 2 * H), lambda i: (i, 0)),
        ],
        out_shape=[
            jax.ShapeDtypeStruct((N, F), jnp.float32),
            jax.ShapeDtypeStruct((N, 2 * H), jnp.float32),
        ],
    )(x, fc_w, fc_b, w8, b8)


def _sc1_body(src3_hbm, dst3_hbm, snode_hbm, zerosn_hbm,
              sf_hbm, dpart_hbm,
              snode_v, src_t, dst_t, sfb, srb, d_sp):
    c = lax.axis_index("c")
    s = lax.axis_index("s")
    cwid = c * NS + s
    base_e = cwid * EB

    pltpu.sync_copy(snode_hbm, snode_v)
    pltpu.sync_copy(src3_hbm.at[cwid], src_t)
    pltpu.sync_copy(dst3_hbm.at[cwid], dst_t)

    @pl.when(s == 0)
    def _():
        for h in range(H):
            pltpu.sync_copy(zerosn_hbm, d_sp[h])

    plsc.subcore_barrier()

    def chunk_body(k, carry):
        e0 = base_e + k * CH
        @plsc.parallel_loop(0, CH, step=16, unroll=G)
        def _groups(g0):
            sb = src_t[k, pl.ds(g0, 16)] * 8
            db = dst_t[k, pl.ds(g0, 16)] * 8
            for h in range(H):
                al_s = plsc.load_gather(snode_v, [sb + h])
                ar_s = plsc.load_gather(snode_v, [sb + (H + h)])
                al_d = plsc.load_gather(snode_v, [db + h])
                ar_d = plsc.load_gather(snode_v, [db + (H + h)])
                sfb[h][pl.ds(g0, 16)] = _lrelu(al_s + ar_d)
                srb[h][pl.ds(g0, 16)] = _lrelu(al_d + ar_s)
        for h in range(H):
            pltpu.sync_copy(sfb[h], sf_hbm.at[pl.ds(h * E + e0, CH)])
            pltpu.sync_copy(sfb[h], d_sp[h].at[src_t.at[k]], add=True)
            pltpu.sync_copy(srb[h], d_sp[h].at[dst_t.at[k]], add=True)
        return carry

    lax.fori_loop(0, NCHUNK, chunk_body, 0)

    plsc.subcore_barrier()

    @pl.when(s == 0)
    def _():
        for h in range(H):
            pltpu.sync_copy(d_sp[h], dpart_hbm.at[c, h])


def _sc1(src3, dst3, snode_flat, zerosn):
    mesh = plsc.VectorSubcoreMesh(core_axis_name="c", subcore_axis_name="s")
    return pl.kernel(
        _sc1_body,
        out_type=[
            jax.ShapeDtypeStruct((H * E,), jnp.float32),
            jax.ShapeDtypeStruct((NC, H, N), jnp.float32),
        ],
        mesh=mesh,
        compiler_params=pltpu.CompilerParams(needs_layout_passes=False),
        scratch_types=[
            pltpu.VMEM((N * 2 * H,), jnp.float32),
            pltpu.VMEM((NCHUNK, CH), jnp.int32),
            pltpu.VMEM((NCHUNK, CH), jnp.int32),
            [pltpu.VMEM((CH,), jnp.float32) for _ in range(H)],
            [pltpu.VMEM((CH,), jnp.float32) for _ in range(H)],
            [pltpu.VMEM_SHARED((N,), jnp.float32) for _ in range(H)],
        ],
    )(src3, dst3, snode_flat, zerosn)


def _sc2_body(src_hbm, sf_hbm, dpart_hbm,
              alpha_hbm,
              dpart_v, src_v, sfc, alc):
    c = lax.axis_index("c")
    s = lax.axis_index("s")
    cwid = c * NS + s
    base_e = cwid * EB

    pltpu.sync_copy(dpart_hbm, dpart_v)

    def chunk_body(k, carry):
        e0 = base_e + k * CH
        pltpu.sync_copy(src_hbm.at[pl.ds(e0, CH)], src_v)
        for h in range(H):
            pltpu.sync_copy(sf_hbm.at[pl.ds(h * E + e0, CH)], sfc[h])
        @plsc.parallel_loop(0, CH, step=16, unroll=G)
        def _groups(g0):
            s16 = src_v[pl.ds(g0, 16)]
            for h in range(H):
                d0 = plsc.load_gather(dpart_v, [s16 + h * N])
                d1 = plsc.load_gather(dpart_v, [s16 + (H * N + h * N)])
                sfv = sfc[h][pl.ds(g0, 16)]
                alc[h][pl.ds(g0, 16)] = sfv / (d0 + d1)
        for h in range(H):
            pltpu.sync_copy(alc[h], alpha_hbm.at[pl.ds(h * E + e0, CH)])
        return carry

    lax.fori_loop(0, NCHUNK, chunk_body, 0)


def _sc2(src, sf, dpart_flat):
    mesh = plsc.VectorSubcoreMesh(core_axis_name="c", subcore_axis_name="s")
    return pl.kernel(
        _sc2_body,
        out_type=jax.ShapeDtypeStruct((H * E,), jnp.float32),
        mesh=mesh,
        compiler_params=pltpu.CompilerParams(needs_layout_passes=False),
        scratch_types=[
            pltpu.VMEM((NC * H * N,), jnp.float32),
            pltpu.VMEM((CH,), jnp.int32),
            [pltpu.VMEM((CH,), jnp.float32) for _ in range(H)],
            [pltpu.VMEM((CH,), jnp.float32) for _ in range(H)],
        ],
    )(src, sf, dpart_flat)


def _sc3_body(hidx, src3_hbm, dst3_hbm, alpha_hbm, fch_hbm,
              agg_hbm,
              src_c, dst_c, af_c, frows, stages, agg_sp, gsems, ssems):
    c = lax.axis_index("c")
    s = lax.axis_index("s")
    cwid = c * NS + s
    base_e = cwid * EB

    def zb(e, zcarry):
        for j in range(F // 16):
            stages[0][e, pl.ds(j * 16, 16)] = jnp.zeros((16,), jnp.float32)
        return zcarry

    lax.fori_loop(0, CH3, zb, 0)

    row0 = s * 624
    for t in range(7):
        pltpu.sync_copy(stages[0], agg_sp.at[pl.ds(row0 + t * CH3, CH3)])
    pltpu.sync_copy(stages[0].at[pl.ds(0, 64)], agg_sp.at[pl.ds(row0 + 560, 64)])

    @pl.when(s == NS - 1)
    def _():
        pltpu.sync_copy(stages[0].at[pl.ds(0, 16)], agg_sp.at[pl.ds(9984, 16)])

    plsc.subcore_barrier()

    def load_chunk(q, b):
        pltpu.sync_copy(src3_hbm.at[cwid, pl.ds(q, 1)], src_c[b])
        pltpu.sync_copy(dst3_hbm.at[cwid, pl.ds(q, 1)], dst_c[b])
        pltpu.sync_copy(alpha_hbm.at[pl.ds(hidx * E + base_e + q * CH3, CH3)],
                        af_c[b])
        pltpu.async_copy(fch_hbm.at[dst_c[b].at[0]], frows[b], gsems[b])

    def wait_gather(b):
        pltpu.make_async_copy(fch_hbm.at[dst_c[b].at[0]], frows[b],
                              gsems[b]).wait()

    def compute(b):
        frow = frows[b]
        stage = stages[b]
        af = af_c[b]

        @plsc.parallel_loop(0, CH3, step=1, unroll=8)
        def edge_body(e):
            av = plsc.load_gather(af, [jnp.full((16,), e, dtype=jnp.int32)])
            for j in range(F // 16):
                stage[e, pl.ds(j * 16, 16)] = av * frow[e, pl.ds(j * 16, 16)]

    def start_scatter(b):
        return pltpu.async_copy(stages[b], agg_sp.at[src_c[b].at[0]], ssems[b],
                                add=True)

    load_chunk(0, 0)

    def pair_body(i, carry):
        qb = 2 * i + 1
        qn = jnp.minimum(2 * i + 2, NCH3 - 1)
        load_chunk(qb, 1)
        wait_gather(0)
        compute(0)
        da = start_scatter(0)
        load_chunk(qn, 0)
        da.wait()
        wait_gather(1)
        compute(1)
        start_scatter(1).wait()
        return carry

    lax.fori_loop(0, (NCH3 - 1) // 2, pair_body, 0)

    wait_gather(0)
    compute(0)
    start_scatter(0).wait()

    plsc.subcore_barrier()

    for t in range(7):
        pltpu.sync_copy(agg_sp.at[pl.ds(row0 + t * CH3, CH3)],
                        agg_hbm.at[c, pl.ds(row0 + t * CH3, CH3)])
    pltpu.sync_copy(agg_sp.at[pl.ds(row0 + 560, 64)],
                    agg_hbm.at[c, pl.ds(row0 + 560, 64)])

    @pl.when(s == NS - 1)
    def _():
        pltpu.sync_copy(agg_sp.at[pl.ds(9984, 16)],
                        agg_hbm.at[c, pl.ds(9984, 16)])


def _sc3(src3b, dst3b, alpha, fc_h, hidx):
    mesh = plsc.VectorSubcoreMesh(core_axis_name="c", subcore_axis_name="s")
    return pl.kernel(
        functools.partial(_sc3_body, hidx),
        out_type=jax.ShapeDtypeStruct((NC, N, F), jnp.float32),
        mesh=mesh,
        compiler_params=pltpu.CompilerParams(needs_layout_passes=False),
        scratch_types=[
            [pltpu.VMEM((1, CH3), jnp.int32) for _ in range(2)],
            [pltpu.VMEM((1, CH3), jnp.int32) for _ in range(2)],
            [pltpu.VMEM((CH3,), jnp.float32) for _ in range(2)],
            [pltpu.VMEM((CH3, F), jnp.float32) for _ in range(2)],
            [pltpu.VMEM((CH3, F), jnp.float32) for _ in range(2)],
            pltpu.VMEM_SHARED((N, F), jnp.float32),
            [pltpu.SemaphoreType.DMA for _ in range(2)],
            [pltpu.SemaphoreType.DMA for _ in range(2)],
        ],
    )(src3b, dst3b, alpha, fc_h)


def _tc2_body(*refs):
    head_refs = refs[:H]
    out_ref = refs[H]
    v = jnp.concatenate([r[:, 0, :] + r[:, 1, :] for r in head_refs], axis=1)
    out_ref[...] = jnp.where(v > 0, v, jnp.exp(v) - 1.0)


def _tc2(agg_heads):
    bn = 1000
    grid = (N // bn,)
    return pl.pallas_call(
        _tc2_body,
        grid=grid,
        in_specs=[pl.BlockSpec((bn, NC, F), lambda i: (i, 0, 0))
                  for _ in range(H)],
        out_specs=pl.BlockSpec((bn, H * F), lambda i: (i, 0)),
        out_shape=jax.ShapeDtypeStruct((N, H * F), jnp.float32),
    )(*agg_heads)


def kernel(x, edge_index, attn_w, attn_b, fc_w, fc_b):
    src = edge_index[0].astype(jnp.int32)
    dst = edge_index[1].astype(jnp.int32)

    w8 = jnp.concatenate([attn_w[:, :F].T, attn_w[:, F:].T], axis=1)
    b8 = jnp.concatenate([attn_b, jnp.zeros((H,), jnp.float32)])

    fc_h, snode = _tc1(x, fc_w, fc_b, w8, b8)

    zerosn = jnp.zeros((N,), jnp.float32)

    src3 = src.reshape(NW, NCHUNK, CH)
    dst3 = dst.reshape(NW, NCHUNK, CH)
    src3b = src.reshape(NW, NCH3, CH3)
    dst3b = dst.reshape(NW, NCH3, CH3)

    sf, dpart = _sc1(src3, dst3, snode.reshape(-1), zerosn)
    alpha = _sc2(src, sf, dpart.reshape(-1))

    agg_heads = []
    for hidx in range(H):
        part = _sc3(src3b, dst3b, alpha, fc_h, hidx)
        agg_heads.append(part.transpose(1, 0, 2))

    return _tc2(agg_heads)
